# baseline (speedup 1.0000x reference)
import functools

import jax
import jax.numpy as jnp
from jax import lax
from jax.experimental import pallas as pl
from jax.experimental.pallas import tpu as pltpu
from jax.experimental.pallas import tpu_sc as plsc

_TAU = 0.67

_NC = 2
_NS = 16
_NW = _NC * _NS
_K = 80


def _make_deg_kernel(e: int, n: int):
    ept = e // _NW
    npad = ((n + 16 * 16 - 1) // (16 * 16)) * 16 * 16
    cpt = npad // _NS
    mesh = plsc.VectorSubcoreMesh(core_axis_name="c", subcore_axis_name="s")

    @functools.partial(
        pl.kernel,
        out_type=jax.ShapeDtypeStruct((2 * npad,), jnp.float32),
        mesh=mesh,
        compiler_params=pltpu.CompilerParams(needs_layout_passes=False),
        scratch_types=[
            pltpu.VMEM((ept,), jnp.int32),
            pltpu.VMEM((npad,), jnp.float32),
            pltpu.VMEM((_NS * cpt,), jnp.float32),
            pltpu.VMEM((cpt,), jnp.float32),
            pltpu.VMEM_SHARED((_NS * npad,), jnp.float32),
        ],
    )
    def deg_kernel(ef_hbm, out_hbm, dst_v, hist_v, red_v, obuf_v, stage_sh):
        c = lax.axis_index("c")
        s = lax.axis_index("s")
        base0 = e + (s * _NC + c) * ept
        pltpu.sync_copy(ef_hbm.at[pl.ds(base0, ept)], dst_v)

        def zbody(i, carry):
            hist_v[pl.ds(i * 16, 16)] = jnp.zeros((16,), jnp.float32)
            return carry

        lax.fori_loop(0, npad // 16, zbody, 0)

        ones = jnp.ones((16,), jnp.float32)

        def body(i, carry):
            idx = dst_v[pl.ds(i * 16, 16)]
            plsc.addupdate_scatter(hist_v, [idx], ones)
            return carry

        lax.fori_loop(0, ept // 16, body, 0)
        pltpu.sync_copy(hist_v, stage_sh.at[pl.ds(s * npad, npad)])
        plsc.subcore_barrier()
        for t in range(_NS):
            pltpu.sync_copy(stage_sh.at[pl.ds(t * npad + s * cpt, cpt)],
                            red_v.at[pl.ds(t * cpt, cpt)])

        def rbody(i, carry):
            v = red_v[pl.ds(i * 16, 16)]
            for t in range(1, _NS):
                v = v + red_v[pl.ds(t * cpt + i * 16, 16)]
            obuf_v[pl.ds(i * 16, 16)] = v
            return carry

        lax.fori_loop(0, cpt // 16, rbody, 0)
        pltpu.sync_copy(obuf_v, out_hbm.at[pl.ds(c * npad + s * cpt, cpt)])

    return deg_kernel


def _make_prop_kernel(e: int, n: int, d: int):
    ept = e // _NW
    kc = 64
    nfull = ept // kc
    kt = ept - nfull * kc
    nbuf = 3
    nbody = nfull // nbuf
    rpt = n // 10
    slab = 64
    mesh = plsc.VectorSubcoreMesh(core_axis_name="c", subcore_axis_name="s")

    @functools.partial(
        pl.kernel,
        out_type=jax.ShapeDtypeStruct((2 * n, d), jnp.float32),
        mesh=mesh,
        scratch_types=[
            pltpu.VMEM((ept,), jnp.int32),
            pltpu.VMEM((ept,), jnp.int32),
            pltpu.VMEM((kc,), jnp.int32),
            pltpu.VMEM((kc,), jnp.int32),
            pltpu.VMEM((kc,), jnp.int32),
            pltpu.VMEM((kc,), jnp.int32),
            pltpu.VMEM((kc,), jnp.int32),
            pltpu.VMEM((kc,), jnp.int32),
            pltpu.VMEM((kc, d), jnp.float32),
            pltpu.VMEM((kc, d), jnp.float32),
            pltpu.VMEM((kc, d), jnp.float32),
            pltpu.VMEM((kt,), jnp.int32),
            pltpu.VMEM((kt,), jnp.int32),
            pltpu.VMEM((kt, d), jnp.float32),
            pltpu.VMEM_SHARED((n, d), jnp.float32),
            pltpu.SemaphoreType.DMA,
            pltpu.SemaphoreType.DMA,
            pltpu.SemaphoreType.DMA,
            pltpu.SemaphoreType.DMA,
            pltpu.SemaphoreType.DMA,
            pltpu.SemaphoreType.DMA,
            pltpu.SemaphoreType.DMA,
        ],
    )
    def prop_kernel(hs_hbm, ef_hbm, out_hbm,
                    srcall_v, dstall_v, sb0, sb1, sb2, db0, db1, db2,
                    rb0, rb1, rb2, srcT, dstT, rowsT, acc_sh,
                    sg0, sg1, sg2, ss0, ss1, ss2, semT):
        c = lax.axis_index("c")
        s = lax.axis_index("s")
        sbs = (sb0, sb1, sb2)
        dbs = (db0, db1, db2)
        rbs = (rb0, rb1, rb2)
        sgs = (sg0, sg1, sg2)
        sss = (ss0, ss1, ss2)
        base0 = (s * _NC + c) * ept
        pltpu.sync_copy(ef_hbm.at[pl.ds(base0, ept)], srcall_v)
        pltpu.sync_copy(ef_hbm.at[pl.ds(e + base0, ept)], dstall_v)

        @pl.when(s < 10)
        def _():
            for t in range(slab * d // 16):
                rb0[t // (d // 16), pl.ds((t % (d // 16)) * 16, 16)] = (
                    jnp.zeros((16,), jnp.float32))
            for r in range(rpt // slab):
                pltpu.sync_copy(rb0, acc_sh.at[pl.ds(s * rpt + r * slab,
                                                     slab)])
            pltpu.sync_copy(rb0.at[pl.ds(0, rpt - (rpt // slab) * slab)],
                            acc_sh.at[pl.ds(s * rpt + (rpt // slab) * slab,
                                            rpt - (rpt // slab) * slab)])

        def _issue_gather(chunk, k):
            for t in range(kc // 16):
                sbs[k][pl.ds(t * 16, 16)] = (
                    srcall_v[pl.ds(chunk * kc + t * 16, 16)])
            pltpu.async_copy(hs_hbm.at[sbs[k]], rbs[k], sgs[k])

        def _gdrain(k):
            pltpu.make_async_copy(hs_hbm.at[pl.ds(0, kc)], rbs[k],
                                  sgs[k]).wait()

        def _sdrain(k):
            pltpu.make_async_copy(rbs[k], acc_sh.at[pl.ds(0, kc)],
                                  sss[k]).wait()

        def _scatter(chunk, k):
            for t in range(kc // 16):
                dbs[k][pl.ds(t * 16, 16)] = (
                    dstall_v[pl.ds(chunk * kc + t * 16, 16)])
            pltpu.async_copy(rbs[k], acc_sh.at[dbs[k]], sss[k], add=True)

        for k in range(nbuf):
            _issue_gather(k, k)
        plsc.subcore_barrier()

        def body(j, carry):
            for k in range(nbuf):
                ch = j * nbuf + k
                _gdrain(k)
                _scatter(ch, k)
                _sdrain(k)
                _issue_gather(ch + nbuf, k)
            return carry

        lax.fori_loop(0, nbody - 1, body, 0)
        for k in range(nbuf):
            ch = (nbody - 1) * nbuf + k
            _gdrain(k)
            _scatter(ch, k)
            _sdrain(k)
        srcT[pl.ds(0, kt)] = srcall_v[pl.ds(nfull * kc, kt)]
        dstT[pl.ds(0, kt)] = dstall_v[pl.ds(nfull * kc, kt)]
        pltpu.async_copy(hs_hbm.at[srcT], rowsT, semT).wait()
        pltpu.sync_copy(rowsT, acc_sh.at[dstT], add=True)

        plsc.subcore_barrier()
        @pl.when(s < 10)
        def _():
            nslab = rpt // slab
            rem = rpt - nslab * slab

            def _odrain(k):
                pltpu.make_async_copy(rbs[k], out_hbm.at[pl.ds(0, slab)],
                                      sss[k]).wait()

            for r in range(nslab):
                k = r % nbuf
                if r >= nbuf:
                    _odrain(k)
                pltpu.sync_copy(acc_sh.at[pl.ds(s * rpt + r * slab, slab)],
                                rbs[k])
                pltpu.async_copy(rbs[k],
                                 out_hbm.at[pl.ds(c * n + s * rpt + r * slab,
                                                  slab)], sss[k])
            _odrain(0)
            pltpu.sync_copy(acc_sh.at[pl.ds(s * rpt + nslab * slab, rem)],
                            rb0.at[pl.ds(0, rem)])
            pltpu.async_copy(rb0.at[pl.ds(0, rem)],
                             out_hbm.at[pl.ds(c * n + s * rpt + nslab * slab,
                                              rem)], sss[0])
            pltpu.make_async_copy(rb0.at[pl.ds(0, rem)],
                                  out_hbm.at[pl.ds(0, rem)], sss[0]).wait()
            _odrain(1)
            _odrain(2)

    return prop_kernel


def _pre_a_body(x_ref, wg1_ref, wm1_ref, bm1_ref, h1_ref, m_ref):
    x = x_ref[...]
    h1_ref[...] = jnp.dot(x, wg1_ref[...], preferred_element_type=jnp.float32)
    m_ref[...] = jnp.maximum(
        jnp.dot(x, wm1_ref[...], preferred_element_type=jnp.float32)
        + bm1_ref[...], 0.0)


def _pre_b_body(degp_ref, h1_ref, hs1_ref, dinv_ref):
    dp = degp_ref[...]
    deg = 1.0 + dp[0:1, :] + dp[1:2, :]
    dinv = lax.rsqrt(deg)
    dinv_ref[...] = dinv
    hs1_ref[...] = h1_ref[...] * jnp.transpose(dinv, (1, 0))


def _mlp_body(m_ref, wm2_ref, bm2_ref, zmlp_ref):
    zmlp_ref[...] = (jnp.dot(m_ref[...], wm2_ref[...],
                             preferred_element_type=jnp.float32)
                     + bm2_ref[...])


def _mid_body(p1_ref, hs1_ref, dinv_ref, bg1_ref, wg2_ref, hs2_ref):
    dinv = jnp.transpose(dinv_ref[...], (1, 0))
    s1 = dinv * (p1_ref[0] + p1_ref[1] + hs1_ref[...]) + bg1_ref[...]
    z = jnp.maximum(s1, 0.0)
    h2 = jnp.dot(z, wg2_ref[...], preferred_element_type=jnp.float32)
    hs2 = h2 * dinv
    hs2_ref[...] = jnp.concatenate([hs2, jnp.zeros_like(hs2)], axis=1)


def _post_body(p2_ref, hs2_ref, dinv_ref, bg2_ref, zmlp_ref, spi_ref,
               logt_ref, out_ref):
    dinv = jnp.transpose(dinv_ref[...], (1, 0))
    d_out = out_ref.shape[1]
    z_gnn = (dinv * (p2_ref[0, :, :d_out] + p2_ref[1, :, :d_out]
                     + hs2_ref[:, :d_out]) + bg2_ref[...])
    u = (spi_ref[...] - _TAU) * jnp.exp(-logt_ref[...])
    beta = 1.0 / (1.0 + jnp.exp(-u))
    out_ref[...] = beta * z_gnn + (1.0 - beta) * zmlp_ref[...]


def _tc_pre_a(x, w_g1, w_m1, b_m1, bm):
    n, d_in = x.shape
    d_hid = w_g1.shape[1]
    grid = (n // bm,)
    return pl.pallas_call(
        _pre_a_body,
        grid=grid,
        in_specs=[
            pl.BlockSpec((bm, d_in), lambda i: (i, 0)),
            pl.BlockSpec((d_in, d_hid), lambda i: (0, 0)),
            pl.BlockSpec((d_in, d_hid), lambda i: (0, 0)),
            pl.BlockSpec((d_hid,), lambda i: (0,)),
        ],
        out_specs=[
            pl.BlockSpec((bm, d_hid), lambda i: (i, 0)),
            pl.BlockSpec((bm, d_hid), lambda i: (i, 0)),
        ],
        out_shape=[
            jax.ShapeDtypeStruct((n, d_hid), jnp.float32),
            jax.ShapeDtypeStruct((n, d_hid), jnp.float32),
        ],
    )(x, w_g1, w_m1, b_m1)


def _tc_pre_b(degp2, h1, bm):
    n, d_hid = h1.shape
    return pl.pallas_call(
        _pre_b_body,
        out_shape=[
            jax.ShapeDtypeStruct((n, d_hid), jnp.float32),
            jax.ShapeDtypeStruct((1, n), jnp.float32),
        ],
    )(degp2, h1)


def _tc_mlp(m, w_m2, b_m2, bm):
    n, d_hid = m.shape
    d_out = w_m2.shape[1]
    grid = (n // bm,)
    return pl.pallas_call(
        _mlp_body,
        grid=grid,
        in_specs=[
            pl.BlockSpec((bm, d_hid), lambda i: (i, 0)),
            pl.BlockSpec((d_hid, d_out), lambda i: (0, 0)),
            pl.BlockSpec((d_out,), lambda i: (0,)),
        ],
        out_specs=pl.BlockSpec((bm, d_out), lambda i: (i, 0)),
        out_shape=jax.ShapeDtypeStruct((n, d_out), jnp.float32),
    )(m, w_m2, b_m2)


def _tc_mid(p1, hs1, dinv, b_g1, w_g2, bm):
    n, d_hid = hs1.shape
    d_out = w_g2.shape[1]
    return pl.pallas_call(
        _mid_body,
        out_shape=jax.ShapeDtypeStruct((n, 2 * d_out), jnp.float32),
    )(p1, hs1, dinv, b_g1, w_g2)


def _tc_post(p2, hs2, dinv, b_g2, zmlp, spi2, logt2, bm):
    n, d_pad = hs2.shape
    d_out = zmlp.shape[1]
    return pl.pallas_call(
        _post_body,
        out_shape=jax.ShapeDtypeStruct((n, d_out), jnp.float32),
    )(p2, hs2, dinv, b_g2, zmlp, spi2, logt2)


def kernel(x, edge_index, spi, W_g1, b_g1, W_g2, b_g2, W_m1, b_m1, W_m2,
           b_m2, log_T):
    n, d_in = x.shape
    e = edge_index.shape[1]
    d_hid = W_g1.shape[1]
    d_out = W_g2.shape[1]
    bm = 2000

    ef = edge_index.reshape(2 * e)

    deg_fn = _make_deg_kernel(e, n)
    degp = deg_fn(ef)
    npad = degp.shape[0] // 2
    degp2 = degp.reshape(2, npad)[:, :n]

    h1, m = _tc_pre_a(x, W_g1, W_m1, b_m1, bm)
    hs1, dinv = _tc_pre_b(degp2, h1, bm)

    prop1 = _make_prop_kernel(e, n, d_hid)
    p1 = prop1(hs1, ef)

    zmlp = _tc_mlp(m, W_m2, b_m2, bm)
    hs2 = _tc_mid(p1.reshape(2, n, d_hid), hs1, dinv, b_g1, W_g2, bm)

    d_pad = hs2.shape[1]
    prop2 = _make_prop_kernel(e, n, d_pad)
    p2 = prop2(hs2, ef)

    return _tc_post(p2.reshape(2, n, d_pad), hs2, dinv, b_g2, zmlp,
                    spi.reshape(1, 1), log_T.reshape(1, 1), bm)


# speedup vs baseline: 1.0071x; 1.0071x over previous
import functools

import jax
import jax.numpy as jnp
from jax import lax
from jax.experimental import pallas as pl
from jax.experimental.pallas import tpu as pltpu
from jax.experimental.pallas import tpu_sc as plsc

_TAU = 0.67

_NC = 2
_NS = 16
_NW = _NC * _NS
_K = 80


def _make_deg_kernel(e: int, n: int):
    ept = e // _NW
    npad = ((n + 16 * 16 - 1) // (16 * 16)) * 16 * 16
    cpt = npad // _NS
    mesh = plsc.VectorSubcoreMesh(core_axis_name="c", subcore_axis_name="s")

    @functools.partial(
        pl.kernel,
        out_type=jax.ShapeDtypeStruct((2 * npad,), jnp.float32),
        mesh=mesh,
        compiler_params=pltpu.CompilerParams(needs_layout_passes=False),
        scratch_types=[
            pltpu.VMEM((2, ept + 112), jnp.int32),
            pltpu.VMEM((ept + 112,), jnp.int32),
            pltpu.VMEM((npad,), jnp.float32),
            pltpu.VMEM((_NS * cpt,), jnp.float32),
            pltpu.VMEM((cpt,), jnp.float32),
            pltpu.VMEM_SHARED((_NS * npad,), jnp.float32),
            pltpu.VMEM_SHARED((_NS * (ept + 112),), jnp.int32),
            pltpu.SemaphoreType.DMA,
        ],
    )
    def deg_kernel(ei_hbm, out_hbm, eib_v, dst_v, hist_v, red_v, obuf_v,
                   stage_sh, bounce_sh, sem):
        c = lax.axis_index("c")
        s = lax.axis_index("s")
        base0 = (s * _NC + c) * ept
        delta = lax.rem(base0, 128)
        start = pl.multiple_of(base0 - delta, 128)
        pltpu.sync_copy(ei_hbm.at[:, pl.ds(start, ept + 112)], eib_v)
        boff = s * (ept + 112)
        pltpu.sync_copy(eib_v.at[1], bounce_sh.at[pl.ds(boff, ept + 112)])
        pltpu.sync_copy(bounce_sh.at[pl.ds(boff, ept + 112)], dst_v)

        def zbody(i, carry):
            hist_v[pl.ds(i * 16, 16)] = jnp.zeros((16,), jnp.float32)
            return carry

        lax.fori_loop(0, npad // 16, zbody, 0)

        ones = jnp.ones((16,), jnp.float32)

        def body(i, carry):
            idx = dst_v[pl.ds(delta + i * 16, 16)]
            plsc.addupdate_scatter(hist_v, [idx], ones)
            return carry

        lax.fori_loop(0, ept // 16, body, 0)
        pltpu.sync_copy(hist_v, stage_sh.at[pl.ds(s * npad, npad)])
        plsc.subcore_barrier()
        for t in range(_NS):
            pltpu.async_copy(stage_sh.at[pl.ds(t * npad + s * cpt, cpt)],
                             red_v.at[pl.ds(t * cpt, cpt)], sem)
        for t in range(_NS):
            pltpu.make_async_copy(stage_sh.at[pl.ds(0, cpt)],
                                  red_v.at[pl.ds(0, cpt)], sem).wait()

        def rbody(i, carry):
            v = red_v[pl.ds(i * 16, 16)]
            for t in range(1, _NS):
                v = v + red_v[pl.ds(t * cpt + i * 16, 16)]
            obuf_v[pl.ds(i * 16, 16)] = v
            return carry

        lax.fori_loop(0, cpt // 16, rbody, 0)
        pltpu.sync_copy(obuf_v, out_hbm.at[pl.ds(c * npad + s * cpt, cpt)])

    return deg_kernel


def _make_prop_kernel(e: int, n: int, d: int):
    ept = e // _NW
    kc = 64
    nfull = ept // kc
    kt = ept - nfull * kc
    nbuf = 3
    nbody = nfull // nbuf
    rpt = n // 10
    slab = 64
    mesh = plsc.VectorSubcoreMesh(core_axis_name="c", subcore_axis_name="s")

    @functools.partial(
        pl.kernel,
        out_type=jax.ShapeDtypeStruct((2 * n, d), jnp.float32),
        mesh=mesh,
        scratch_types=[
            pltpu.VMEM((ept,), jnp.int32),
            pltpu.VMEM((ept,), jnp.int32),
            pltpu.VMEM((kc,), jnp.int32),
            pltpu.VMEM((kc,), jnp.int32),
            pltpu.VMEM((kc,), jnp.int32),
            pltpu.VMEM((kc,), jnp.int32),
            pltpu.VMEM((kc,), jnp.int32),
            pltpu.VMEM((kc,), jnp.int32),
            pltpu.VMEM((kc, d), jnp.float32),
            pltpu.VMEM((kc, d), jnp.float32),
            pltpu.VMEM((kc, d), jnp.float32),
            pltpu.VMEM((kt,), jnp.int32),
            pltpu.VMEM((kt,), jnp.int32),
            pltpu.VMEM((kt, d), jnp.float32),
            pltpu.VMEM_SHARED((n, d), jnp.float32),
            pltpu.SemaphoreType.DMA,
            pltpu.SemaphoreType.DMA,
            pltpu.SemaphoreType.DMA,
            pltpu.SemaphoreType.DMA,
            pltpu.SemaphoreType.DMA,
            pltpu.SemaphoreType.DMA,
            pltpu.SemaphoreType.DMA,
        ],
    )
    def prop_kernel(hs_hbm, ef_hbm, out_hbm,
                    srcall_v, dstall_v, sb0, sb1, sb2, db0, db1, db2,
                    rb0, rb1, rb2, srcT, dstT, rowsT, acc_sh,
                    sg0, sg1, sg2, ss0, ss1, ss2, semT):
        c = lax.axis_index("c")
        s = lax.axis_index("s")
        sbs = (sb0, sb1, sb2)
        dbs = (db0, db1, db2)
        rbs = (rb0, rb1, rb2)
        sgs = (sg0, sg1, sg2)
        sss = (ss0, ss1, ss2)
        base0 = (s * _NC + c) * ept
        pltpu.sync_copy(ef_hbm.at[pl.ds(base0, ept)], srcall_v)
        pltpu.sync_copy(ef_hbm.at[pl.ds(e + base0, ept)], dstall_v)

        @pl.when(s < 10)
        def _():
            for t in range(slab * d // 16):
                rb0[t // (d // 16), pl.ds((t % (d // 16)) * 16, 16)] = (
                    jnp.zeros((16,), jnp.float32))
            for r in range(rpt // slab):
                pltpu.sync_copy(rb0, acc_sh.at[pl.ds(s * rpt + r * slab,
                                                     slab)])
            pltpu.sync_copy(rb0.at[pl.ds(0, rpt - (rpt // slab) * slab)],
                            acc_sh.at[pl.ds(s * rpt + (rpt // slab) * slab,
                                            rpt - (rpt // slab) * slab)])

        def _issue_gather(chunk, k):
            for t in range(kc // 16):
                sbs[k][pl.ds(t * 16, 16)] = (
                    srcall_v[pl.ds(chunk * kc + t * 16, 16)])
            pltpu.async_copy(hs_hbm.at[sbs[k]], rbs[k], sgs[k])

        def _gdrain(k):
            pltpu.make_async_copy(hs_hbm.at[pl.ds(0, kc)], rbs[k],
                                  sgs[k]).wait()

        def _sdrain(k):
            pltpu.make_async_copy(rbs[k], acc_sh.at[pl.ds(0, kc)],
                                  sss[k]).wait()

        def _scatter(chunk, k):
            for t in range(kc // 16):
                dbs[k][pl.ds(t * 16, 16)] = (
                    dstall_v[pl.ds(chunk * kc + t * 16, 16)])
            pltpu.async_copy(rbs[k], acc_sh.at[dbs[k]], sss[k], add=True)

        for k in range(nbuf):
            _issue_gather(k, k)
        plsc.subcore_barrier()

        def body(j, carry):
            for k in range(nbuf):
                ch = j * nbuf + k
                _gdrain(k)
                _scatter(ch, k)
                _sdrain(k)
                _issue_gather(ch + nbuf, k)
            return carry

        lax.fori_loop(0, nbody - 1, body, 0)
        for k in range(nbuf):
            ch = (nbody - 1) * nbuf + k
            _gdrain(k)
            _scatter(ch, k)
            _sdrain(k)
        srcT[pl.ds(0, kt)] = srcall_v[pl.ds(nfull * kc, kt)]
        dstT[pl.ds(0, kt)] = dstall_v[pl.ds(nfull * kc, kt)]
        pltpu.async_copy(hs_hbm.at[srcT], rowsT, semT).wait()
        pltpu.sync_copy(rowsT, acc_sh.at[dstT], add=True)

        plsc.subcore_barrier()
        @pl.when(s < 10)
        def _():
            nslab = rpt // slab
            rem = rpt - nslab * slab

            def _odrain(k):
                pltpu.make_async_copy(rbs[k], out_hbm.at[pl.ds(0, slab)],
                                      sss[k]).wait()

            for r in range(nslab):
                k = r % nbuf
                if r >= nbuf:
                    _odrain(k)
                pltpu.sync_copy(acc_sh.at[pl.ds(s * rpt + r * slab, slab)],
                                rbs[k])
                pltpu.async_copy(rbs[k],
                                 out_hbm.at[pl.ds(c * n + s * rpt + r * slab,
                                                  slab)], sss[k])
            _odrain(0)
            pltpu.sync_copy(acc_sh.at[pl.ds(s * rpt + nslab * slab, rem)],
                            rb0.at[pl.ds(0, rem)])
            pltpu.async_copy(rb0.at[pl.ds(0, rem)],
                             out_hbm.at[pl.ds(c * n + s * rpt + nslab * slab,
                                              rem)], sss[0])
            pltpu.make_async_copy(rb0.at[pl.ds(0, rem)],
                                  out_hbm.at[pl.ds(0, rem)], sss[0]).wait()
            _odrain(1)
            _odrain(2)

    return prop_kernel


def _pre_a_body(x_ref, wg1_ref, wm1_ref, bm1_ref, h1_ref, m_ref):
    x = x_ref[...]
    h1_ref[...] = jnp.dot(x, wg1_ref[...], preferred_element_type=jnp.float32)
    m_ref[...] = jnp.maximum(
        jnp.dot(x, wm1_ref[...], preferred_element_type=jnp.float32)
        + bm1_ref[...], 0.0)


def _pre_b_body(degp_ref, h1_ref, hs1_ref, dinv_ref):
    dp = degp_ref[...]
    deg = 1.0 + dp[0:1, :] + dp[1:2, :]
    dinv = lax.rsqrt(deg)
    dinv_ref[...] = dinv
    hs1_ref[...] = h1_ref[...] * jnp.transpose(dinv, (1, 0))


def _mlp_body(m_ref, wm2_ref, bm2_ref, zmlp_ref):
    zmlp_ref[...] = (jnp.dot(m_ref[...], wm2_ref[...],
                             preferred_element_type=jnp.float32)
                     + bm2_ref[...])


def _mid_body(p1_ref, hs1_ref, dinv_ref, bg1_ref, wg2_ref, hs2_ref):
    dinv = jnp.transpose(dinv_ref[...], (1, 0))
    s1 = dinv * (p1_ref[0] + p1_ref[1] + hs1_ref[...]) + bg1_ref[...]
    z = jnp.maximum(s1, 0.0)
    h2 = jnp.dot(z, wg2_ref[...], preferred_element_type=jnp.float32)
    hs2 = h2 * dinv
    hs2_ref[...] = jnp.concatenate([hs2, jnp.zeros_like(hs2)], axis=1)


def _post_body(p2_ref, hs2_ref, dinv_ref, bg2_ref, zmlp_ref, spi_ref,
               logt_ref, out_ref):
    dinv = jnp.transpose(dinv_ref[...], (1, 0))
    d_out = out_ref.shape[1]
    z_gnn = (dinv * (p2_ref[0, :, :d_out] + p2_ref[1, :, :d_out]
                     + hs2_ref[:, :d_out]) + bg2_ref[...])
    u = (spi_ref[...] - _TAU) * jnp.exp(-logt_ref[...])
    beta = 1.0 / (1.0 + jnp.exp(-u))
    out_ref[...] = beta * z_gnn + (1.0 - beta) * zmlp_ref[...]


def _tc_pre_a(x, w_g1, w_m1, b_m1, bm):
    n, d_in = x.shape
    d_hid = w_g1.shape[1]
    grid = (n // bm,)
    return pl.pallas_call(
        _pre_a_body,
        grid=grid,
        in_specs=[
            pl.BlockSpec((bm, d_in), lambda i: (i, 0)),
            pl.BlockSpec((d_in, d_hid), lambda i: (0, 0)),
            pl.BlockSpec((d_in, d_hid), lambda i: (0, 0)),
            pl.BlockSpec((d_hid,), lambda i: (0,)),
        ],
        out_specs=[
            pl.BlockSpec((bm, d_hid), lambda i: (i, 0)),
            pl.BlockSpec((bm, d_hid), lambda i: (i, 0)),
        ],
        out_shape=[
            jax.ShapeDtypeStruct((n, d_hid), jnp.float32),
            jax.ShapeDtypeStruct((n, d_hid), jnp.float32),
        ],
    )(x, w_g1, w_m1, b_m1)


def _tc_pre_b(degp2, h1, bm):
    n, d_hid = h1.shape
    return pl.pallas_call(
        _pre_b_body,
        out_shape=[
            jax.ShapeDtypeStruct((n, d_hid), jnp.float32),
            jax.ShapeDtypeStruct((1, n), jnp.float32),
        ],
    )(degp2, h1)


def _tc_mlp(m, w_m2, b_m2, bm):
    n, d_hid = m.shape
    d_out = w_m2.shape[1]
    grid = (n // bm,)
    return pl.pallas_call(
        _mlp_body,
        grid=grid,
        in_specs=[
            pl.BlockSpec((bm, d_hid), lambda i: (i, 0)),
            pl.BlockSpec((d_hid, d_out), lambda i: (0, 0)),
            pl.BlockSpec((d_out,), lambda i: (0,)),
        ],
        out_specs=pl.BlockSpec((bm, d_out), lambda i: (i, 0)),
        out_shape=jax.ShapeDtypeStruct((n, d_out), jnp.float32),
    )(m, w_m2, b_m2)


def _tc_mid(p1, hs1, dinv, b_g1, w_g2, bm):
    n, d_hid = hs1.shape
    d_out = w_g2.shape[1]
    return pl.pallas_call(
        _mid_body,
        out_shape=jax.ShapeDtypeStruct((n, 2 * d_out), jnp.float32),
    )(p1, hs1, dinv, b_g1, w_g2)


def _tc_post(p2, hs2, dinv, b_g2, zmlp, spi2, logt2, bm):
    n, d_pad = hs2.shape
    d_out = zmlp.shape[1]
    return pl.pallas_call(
        _post_body,
        out_shape=jax.ShapeDtypeStruct((n, d_out), jnp.float32),
    )(p2, hs2, dinv, b_g2, zmlp, spi2, logt2)


def kernel(x, edge_index, spi, W_g1, b_g1, W_g2, b_g2, W_m1, b_m1, W_m2,
           b_m2, log_T):
    n, d_in = x.shape
    e = edge_index.shape[1]
    d_hid = W_g1.shape[1]
    d_out = W_g2.shape[1]
    bm = 2000

    ef = edge_index.reshape(2 * e)

    deg_fn = _make_deg_kernel(e, n)
    degp = deg_fn(edge_index)
    npad = degp.shape[0] // 2
    degp2 = degp.reshape(2, npad)[:, :n]

    h1, m = _tc_pre_a(x, W_g1, W_m1, b_m1, bm)
    hs1, dinv = _tc_pre_b(degp2, h1, bm)

    prop1 = _make_prop_kernel(e, n, d_hid)
    p1 = prop1(hs1, ef)

    zmlp = _tc_mlp(m, W_m2, b_m2, bm)
    hs2 = _tc_mid(p1.reshape(2, n, d_hid), hs1, dinv, b_g1, W_g2, bm)

    d_pad = hs2.shape[1]
    prop2 = _make_prop_kernel(e, n, d_pad)
    p2 = prop2(hs2, ef)

    return _tc_post(p2.reshape(2, n, d_pad), hs2, dinv, b_g2, zmlp,
                    spi.reshape(1, 1), log_T.reshape(1, 1), bm)
